# baseline (speedup 1.0000x reference)
import functools

import jax
import jax.numpy as jnp
from jax import lax
from jax.experimental import pallas as pl
from jax.experimental.pallas import tpu as pltpu
from jax.experimental.pallas import tpu_sc as plsc

D = 128
B = 4096 * 50
NC, NS = 2, 16
NW = NC * NS
BPW = B // NW
CH = 128
NCHUNK = BPW // CH
NBUF = 5
OUTER = NCHUNK // NBUF

G = 1024
PB = B // G

_mesh = plsc.VectorSubcoreMesh(core_axis_name="c", subcore_axis_name="s")


@functools.partial(
    pl.kernel,
    mesh=_mesh,
    out_type=jax.ShapeDtypeStruct((B, D), jnp.float32),
    scratch_types=(
        [pltpu.VMEM((BPW,), jnp.int32)]
        + [pltpu.VMEM((CH, D), jnp.float32)] * NBUF
        + [pltpu.SemaphoreType.DMA] * (2 * NBUF)
    ),
)
def _concept_sc(cidx_hbm, ctab_hbm, out_hbm, idx_v, *bufs_and_sems):
    rows = bufs_and_sems[:NBUF]
    gsem = bufs_and_sems[NBUF:2 * NBUF]
    ssem = bufs_and_sems[2 * NBUF:]

    wid = lax.axis_index("s") * NC + lax.axis_index("c")
    base = wid * BPW

    pltpu.sync_copy(cidx_hbm.at[pl.ds(base, BPW)], idx_v)

    for b in range(NBUF):
        pltpu.async_copy(ctab_hbm.at[idx_v.at[pl.ds(b * CH, CH)]],
                         rows[b], gsem[b])

    def outer(k, carry):
        for b in range(NBUF):
            i = k * NBUF + b
            off = i * CH
            pltpu.make_async_copy(ctab_hbm.at[pl.ds(0, CH)],
                                  rows[b], gsem[b]).wait()
            pltpu.async_copy(rows[b], out_hbm.at[pl.ds(base + off, CH)],
                             ssem[b])
            pltpu.make_async_copy(rows[b], out_hbm.at[pl.ds(0, CH)],
                                  ssem[b]).wait()
            nxt = i + NBUF

            @pl.when(nxt < NCHUNK)
            def _fire():
                pltpu.async_copy(
                    ctab_hbm.at[idx_v.at[pl.ds(nxt * CH, CH)]],
                    rows[b], gsem[b])
        return carry

    lax.fori_loop(0, OUTER, outer, 0)


def _tc_body(ridx_ref, tpad_ref, cin_ref, cout_ref, rout_ref, scr_ref):
    idx = ridx_ref[0]
    idxb = jnp.broadcast_to(idx, (D, PB))
    kio = lax.broadcasted_iota(jnp.int32, (D, PB), 0)
    ohT = (kio == idxb).astype(jnp.float32)
    rows = lax.dot_general(ohT, tpad_ref[...],
                           (((0,), (0,)), ((), ())),
                           precision=lax.Precision.HIGHEST,
                           preferred_element_type=jnp.float32)
    scr_ref[...] = rows
    for j in range(4):
        cout_ref[0, j] = cin_ref[0, pl.ds(j * 50, 50), :]
        rout_ref[0, j] = scr_ref[pl.ds(j * 50, 50), :]


_relay_tc = pl.pallas_call(
    _tc_body,
    grid=(G,),
    in_specs=[
        pl.BlockSpec((1, 1, PB), lambda i: (i, 0, 0)),
        pl.BlockSpec((D, D), lambda i: (0, 0)),
        pl.BlockSpec((1, PB, D), lambda i: (i, 0, 0)),
    ],
    out_specs=[
        pl.BlockSpec((1, 4, 50, D), lambda i: (i, 0, 0, 0)),
        pl.BlockSpec((1, 4, 50, D), lambda i: (i, 0, 0, 0)),
    ],
    out_shape=[
        jax.ShapeDtypeStruct((G, 4, 50, D), jnp.float32),
        jax.ShapeDtypeStruct((G, 4, 50, D), jnp.float32),
    ],
    scratch_shapes=[pltpu.VMEM((PB, D), jnp.float32)],
)


def kernel(concept_inp, relation_inp, concept_table, relation_table):
    shp = concept_inp.shape
    cidx = concept_inp.reshape(-1).astype(jnp.int32)
    ridx3 = relation_inp.reshape(G, 1, PB).astype(jnp.int32)
    tpad = jnp.pad(relation_table, ((0, D - relation_table.shape[0]), (0, 0)))
    c_lin = _concept_sc(cidx, concept_table)
    cout4, rout4 = _relay_tc(ridx3, tpad, c_lin.reshape(G, PB, D))
    return cout4.reshape(*shp, D), rout4.reshape(*shp, D)


# speedup vs baseline: 2.1516x; 2.1516x over previous
import functools

import jax
import jax.numpy as jnp
from jax import lax
from jax.experimental import pallas as pl
from jax.experimental.pallas import tpu as pltpu
from jax.experimental.pallas import tpu_sc as plsc

D = 128
NB = 4096
S = 50
B = NB * S
NC, NS = 2, 16
NW = NC * NS
BPW = B // NW
CH = 128
NCHUNK = BPW // CH
NBUF = 5
OUTER = NCHUNK // NBUF

G = 256
BB = NB // G
PB = B // G

_mesh = plsc.VectorSubcoreMesh(core_axis_name="c", subcore_axis_name="s")


@functools.partial(
    pl.kernel,
    mesh=_mesh,
    out_type=jax.ShapeDtypeStruct((B, D), jnp.float32),
    scratch_types=(
        [pltpu.VMEM((BPW,), jnp.int32)]
        + [pltpu.VMEM((CH, D), jnp.float32)] * NBUF
        + [pltpu.SemaphoreType.DMA] * (2 * NBUF)
    ),
)
def _concept_sc(cidx_hbm, ctab_hbm, out_hbm, idx_v, *bufs_and_sems):
    rows = bufs_and_sems[:NBUF]
    gsem = bufs_and_sems[NBUF:2 * NBUF]
    ssem = bufs_and_sems[2 * NBUF:]

    wid = lax.axis_index("s") * NC + lax.axis_index("c")
    base = wid * BPW

    pltpu.sync_copy(cidx_hbm.at[pl.ds(base, BPW)], idx_v)

    for b in range(NBUF):
        pltpu.async_copy(ctab_hbm.at[idx_v.at[pl.ds(b * CH, CH)]],
                         rows[b], gsem[b])

    def outer(k, carry):
        for b in range(NBUF):
            i = k * NBUF + b
            off = i * CH
            pltpu.make_async_copy(ctab_hbm.at[pl.ds(0, CH)],
                                  rows[b], gsem[b]).wait()
            pltpu.async_copy(rows[b], out_hbm.at[pl.ds(base + off, CH)],
                             ssem[b])
            pltpu.make_async_copy(rows[b], out_hbm.at[pl.ds(0, CH)],
                                  ssem[b]).wait()
            nxt = i + NBUF

            @pl.when(nxt < NCHUNK)
            def _fire():
                pltpu.async_copy(
                    ctab_hbm.at[idx_v.at[pl.ds(nxt * CH, CH)]],
                    rows[b], gsem[b])
        return carry

    lax.fori_loop(0, OUTER, outer, 0)


def _tc_body(ridx_ref, thi_ref, tlo_ref, cin_ref, cout_ref, rout_ref,
             scr_ref):
    idx = ridx_ref[0]
    idxb = jnp.broadcast_to(idx, (D, PB))
    kio = lax.broadcasted_iota(jnp.int32, (D, PB), 0)
    ohT = (kio == idxb).astype(jnp.bfloat16)
    dn = (((0,), (0,)), ((), ()))
    rows = (lax.dot_general(ohT, thi_ref[...], dn,
                            preferred_element_type=jnp.float32)
            + lax.dot_general(ohT, tlo_ref[...], dn,
                              preferred_element_type=jnp.float32))
    scr_ref[...] = rows
    for j in range(BB):
        cout_ref[j] = cin_ref[pl.ds(j * S, S), :]
        rout_ref[j] = scr_ref[pl.ds(j * S, S), :]


_relay_tc = pl.pallas_call(
    _tc_body,
    grid=(G,),
    in_specs=[
        pl.BlockSpec((1, 1, PB), lambda i: (i, 0, 0)),
        pl.BlockSpec((D, D), lambda i: (0, 0)),
        pl.BlockSpec((D, D), lambda i: (0, 0)),
        pl.BlockSpec((PB, D), lambda i: (i, 0)),
    ],
    out_specs=[
        pl.BlockSpec((BB, S, D), lambda i: (i, 0, 0)),
        pl.BlockSpec((BB, S, D), lambda i: (i, 0, 0)),
    ],
    out_shape=[
        jax.ShapeDtypeStruct((NB, S, D), jnp.float32),
        jax.ShapeDtypeStruct((NB, S, D), jnp.float32),
    ],
    scratch_shapes=[pltpu.VMEM((PB, D), jnp.float32)],
)


def kernel(concept_inp, relation_inp, concept_table, relation_table):
    cidx = concept_inp.reshape(-1).astype(jnp.int32)
    ridx3 = relation_inp.reshape(G, 1, PB).astype(jnp.int32)
    tpad = jnp.pad(relation_table, ((0, D - relation_table.shape[0]), (0, 0)))
    thi = tpad.astype(jnp.bfloat16)
    tlo = (tpad - thi.astype(jnp.float32)).astype(jnp.bfloat16)
    c_lin = _concept_sc(cidx, concept_table)
    cout, rout = _relay_tc(ridx3, thi, tlo, c_lin)
    return cout, rout
